# Initial kernel scaffold; baseline (speedup 1.0000x reference)
import jax
import jax.numpy as jnp
from jax.experimental import pallas as pl


def kernel(feat, edge_index, W_neigh, b_neigh, W_self):
    raise NotImplementedError("write your pallas kernel here")


# speedup vs baseline: 8.8308x; 8.8308x over previous
import functools

import jax
import jax.numpy as jnp
from jax import lax
from jax.experimental import pallas as pl
from jax.experimental.pallas import tpu as pltpu
from jax.experimental.pallas import tpu_sc as plsc

N_NODES = 10000
D = 128
DH = D // 2

NC = 2
NS = 16
CHUNK = 128

N_PAD = 10112
ZROWS = N_PAD // NS


def _make_agg(n_chunks):
    mesh = plsc.VectorSubcoreMesh(core_axis_name="c", subcore_axis_name="s")

    @functools.partial(
        pl.kernel,
        out_type=jax.ShapeDtypeStruct((NC, N_PAD, DH), jnp.float32),
        mesh=mesh,
        compiler_params=pltpu.CompilerParams(use_tc_tiling_on_sc=False),
        scratch_types=[
            pltpu.VMEM((n_chunks, CHUNK), jnp.int32),
            pltpu.VMEM((n_chunks, CHUNK), jnp.int32),
            pltpu.VMEM((CHUNK, DH), jnp.float32),
            pltpu.VMEM((CHUNK, DH), jnp.float32),
            pltpu.VMEM_SHARED((N_PAD, DH), jnp.float32),
            pltpu.SemaphoreType.DMA,
            pltpu.SemaphoreType.DMA,
        ],
    )
    def agg(src_hbm, dst_hbm, feat_hbm, zeros_hbm, out_hbm,
            src_v, dst_v, rows_a, rows_b, acc, sem_a, sem_b):
        cid = lax.axis_index("c")
        sid = lax.axis_index("s")

        pltpu.sync_copy(zeros_hbm, acc.at[pl.ds(sid * ZROWS, ZROWS)])
        pltpu.sync_copy(src_hbm.at[sid], src_v)
        pltpu.sync_copy(dst_hbm.at[sid], dst_v)

        def scale(j, _):
            for g in range(CHUNK // 16):
                s = src_v[j, pl.ds(g * 16, 16)]
                src_v[j, pl.ds(g * 16, 16)] = s + s + cid
            return 0

        lax.fori_loop(0, n_chunks, scale, 0)
        plsc.subcore_barrier()

        def body(i, _):
            ja = 2 * i
            jb = 2 * i + 1
            da = pltpu.async_copy(feat_hbm.at[src_v.at[ja]], rows_a, sem_a)
            db = pltpu.async_copy(feat_hbm.at[src_v.at[jb]], rows_b, sem_b)
            da.wait()
            pltpu.sync_copy(rows_a, acc.at[dst_v.at[ja]], add=True)
            db.wait()
            pltpu.sync_copy(rows_b, acc.at[dst_v.at[jb]], add=True)
            return 0

        lax.fori_loop(0, n_chunks // 2, body, 0)
        plsc.subcore_barrier()

        pltpu.sync_copy(
            acc.at[pl.ds(sid * ZROWS, ZROWS)],
            out_hbm.at[cid, pl.ds(sid * ZROWS, ZROWS)],
        )

    return agg


def _dense_body(acc_ref, feat_ref, wn_ref, ws_ref, b_ref, out_ref):
    out_ref[...] = (
        jnp.dot(acc_ref[0], wn_ref[0:DH, :], preferred_element_type=jnp.float32)
        + jnp.dot(acc_ref[1], wn_ref[DH:D, :], preferred_element_type=jnp.float32)
        + jnp.dot(feat_ref[...], ws_ref[...], preferred_element_type=jnp.float32)
        + b_ref[...]
    )


def _make_dense(blk, n_blk):
    return pl.pallas_call(
        _dense_body,
        grid=(n_blk,),
        in_specs=[
            pl.BlockSpec((NC, blk, DH), lambda i: (0, i, 0)),
            pl.BlockSpec((blk, D), lambda i: (i, 0)),
            pl.BlockSpec((D, D), lambda i: (0, 0)),
            pl.BlockSpec((D, D), lambda i: (0, 0)),
            pl.BlockSpec((1, D), lambda i: (0, 0)),
        ],
        out_specs=pl.BlockSpec((blk, D), lambda i: (i, 0)),
        out_shape=jax.ShapeDtypeStruct((N_NODES, D), jnp.float32),
    )


def kernel(feat, edge_index, W_neigh, b_neigh, W_self):
    src = edge_index[0].astype(jnp.int32)
    dst = edge_index[1].astype(jnp.int32)
    n_edges = src.shape[0]

    per_tile = -(-n_edges // NS)
    n_chunks = -(-per_tile // CHUNK)
    if n_chunks % 2:
        n_chunks += 1
    e_pad = NS * n_chunks * CHUNK

    pad = e_pad - n_edges
    pad_idx = jnp.arange(pad, dtype=jnp.int32)
    src_p = jnp.concatenate([src, pad_idx % N_NODES]).reshape(NS, n_chunks, CHUNK)
    dst_p = jnp.concatenate(
        [dst, N_NODES + (pad_idx & 63)]
    ).reshape(NS, n_chunks, CHUNK)

    feat_half = feat.reshape(2 * N_NODES, DH)
    zeros = jnp.zeros((ZROWS, DH), jnp.float32)
    acc = _make_agg(n_chunks)(src_p, dst_p, feat_half, zeros)

    blk = 1000
    n_blk = N_NODES // blk
    return _make_dense(blk, n_blk)(acc, feat, W_neigh, W_self,
                                   b_neigh.reshape(1, D))


# speedup vs baseline: 9.4360x; 1.0685x over previous
import functools

import jax
import jax.numpy as jnp
from jax import lax
from jax.experimental import pallas as pl
from jax.experimental.pallas import tpu as pltpu
from jax.experimental.pallas import tpu_sc as plsc

N_NODES = 10000
D = 128
DH = D // 2

NC = 2
NS = 16
CHUNK = 128
NBUF = 4

N_PAD = 10112
ZROWS = N_PAD // NS


def _make_agg(n_chunks):
    mesh = plsc.VectorSubcoreMesh(core_axis_name="c", subcore_axis_name="s")

    @functools.partial(
        pl.kernel,
        out_type=jax.ShapeDtypeStruct((NC, N_PAD, DH), jnp.float32),
        mesh=mesh,
        compiler_params=pltpu.CompilerParams(use_tc_tiling_on_sc=False),
        scratch_types=[
            pltpu.VMEM((n_chunks, CHUNK), jnp.int32),
            pltpu.VMEM((n_chunks, CHUNK), jnp.int32),
            [pltpu.VMEM((CHUNK, DH), jnp.float32) for _ in range(NBUF)],
            pltpu.VMEM_SHARED((N_PAD, DH), jnp.float32),
            [pltpu.SemaphoreType.DMA for _ in range(NBUF)],
            [pltpu.SemaphoreType.DMA for _ in range(NBUF)],
        ],
    )
    def agg(src_hbm, dst_hbm, feat_hbm, zeros_hbm, out_hbm,
            src_v, dst_v, rows, acc, gsem, ssem):
        cid = lax.axis_index("c")
        sid = lax.axis_index("s")

        pltpu.sync_copy(zeros_hbm, acc.at[pl.ds(sid * ZROWS, ZROWS)])
        pltpu.sync_copy(src_hbm.at[cid, sid], src_v)
        pltpu.sync_copy(dst_hbm.at[sid], dst_v)
        plsc.subcore_barrier()

        def body(i, _):
            base = NBUF * i
            gs = [
                pltpu.async_copy(
                    feat_hbm.at[src_v.at[base + k]], rows[k], gsem[k])
                for k in range(NBUF)
            ]
            ss = []
            for k in range(NBUF):
                gs[k].wait()
                ss.append(pltpu.async_copy(
                    rows[k], acc.at[dst_v.at[base + k]], ssem[k],
                    add=True))
            for k in range(NBUF):
                ss[k].wait()
            return 0

        lax.fori_loop(0, n_chunks // NBUF, body, 0)
        plsc.subcore_barrier()

        pltpu.sync_copy(
            acc.at[pl.ds(sid * ZROWS, ZROWS)],
            out_hbm.at[cid, pl.ds(sid * ZROWS, ZROWS)],
        )

    return agg


def _dense_body(acc_ref, feat_ref, wn_ref, ws_ref, b_ref, out_ref):
    out_ref[...] = (
        jnp.dot(acc_ref[0], wn_ref[0:DH, :], preferred_element_type=jnp.float32)
        + jnp.dot(acc_ref[1], wn_ref[DH:D, :], preferred_element_type=jnp.float32)
        + jnp.dot(feat_ref[...], ws_ref[...], preferred_element_type=jnp.float32)
        + b_ref[...]
    )


def _make_dense(blk, n_blk):
    return pl.pallas_call(
        _dense_body,
        grid=(n_blk,),
        in_specs=[
            pl.BlockSpec((NC, blk, DH), lambda i: (0, i, 0)),
            pl.BlockSpec((blk, D), lambda i: (i, 0)),
            pl.BlockSpec((D, D), lambda i: (0, 0)),
            pl.BlockSpec((D, D), lambda i: (0, 0)),
            pl.BlockSpec((1, D), lambda i: (0, 0)),
        ],
        out_specs=pl.BlockSpec((blk, D), lambda i: (i, 0)),
        out_shape=jax.ShapeDtypeStruct((N_NODES, D), jnp.float32),
    )


def kernel(feat, edge_index, W_neigh, b_neigh, W_self):
    src = edge_index[0].astype(jnp.int32)
    dst = edge_index[1].astype(jnp.int32)
    n_edges = src.shape[0]

    per_tile = -(-n_edges // NS)
    n_chunks = -(-per_tile // (CHUNK * NBUF)) * NBUF
    e_pad = NS * n_chunks * CHUNK

    pad = e_pad - n_edges
    pad_idx = jnp.arange(pad, dtype=jnp.int32)
    src_p = jnp.concatenate([src, pad_idx % N_NODES])
    src_all = jnp.stack([2 * src_p, 2 * src_p + 1]).reshape(
        NC, NS, n_chunks, CHUNK)
    dst_p = jnp.concatenate(
        [dst, N_NODES + (pad_idx & 63)]
    ).reshape(NS, n_chunks, CHUNK)

    feat_half = feat.reshape(2 * N_NODES, DH)
    zeros = jnp.zeros((ZROWS, DH), jnp.float32)
    acc = _make_agg(n_chunks)(src_all, dst_p, feat_half, zeros)

    blk = 1000
    n_blk = N_NODES // blk
    return _make_dense(blk, n_blk)(acc, feat, W_neigh, W_self,
                                   b_neigh.reshape(1, D))
